# Initial kernel scaffold; baseline (speedup 1.0000x reference)
import jax
import jax.numpy as jnp
from jax.experimental import pallas as pl


def kernel(x, x_nb):
    raise NotImplementedError("write your pallas kernel here")


# speedup vs baseline: 1.2279x; 1.2279x over previous
import functools

import jax
import jax.numpy as jnp
from jax import lax
from jax.experimental import pallas as pl
from jax.experimental.pallas import tpu as pltpu
from jax.experimental.pallas import tpu_sc as plsc

_N = 10000
_M = 32
_D = 128
_NC = 2
_NS = 16
_NW = _NC * _NS
_NPAD = 10240
_R = _NPAD * _M
_RW = _R // _NW
_CH = 128
_G = _RW // _CH
_NBLK = 256
_EPS = 1e-12


def _prep_body(x_ref, nb_ref, xn_ref, idx_ref):
    xb = x_ref[...]
    nrm = jnp.sqrt(jnp.sum(xb * xb, axis=1, keepdims=True))
    xn_ref[...] = xb / jnp.maximum(nrm, _EPS)
    idx_ref[...] = nb_ref[...] - 1


def _prep(x_pad, nb_pad):
    return pl.pallas_call(
        _prep_body,
        out_shape=[
            jax.ShapeDtypeStruct((_NPAD, _D), jnp.float32),
            jax.ShapeDtypeStruct((_NPAD, _M), jnp.int32),
        ],
    )(x_pad, nb_pad)


def _sc_gather_body(xn_hbm, idx_hbm, z_hbm, idx_v, zbuf, sem):
    wid = lax.axis_index("s") * _NC + lax.axis_index("c")

    def step(g, carry):
        off = wid * _RW + g * _CH
        pltpu.sync_copy(idx_hbm.at[pl.ds(off, _CH)], idx_v.at[0])
        pltpu.async_copy(xn_hbm.at[idx_v.at[0]], zbuf.at[0], sem).wait()
        pltpu.sync_copy(zbuf.at[0], z_hbm.at[pl.ds(off, _CH)])
        return carry

    lax.fori_loop(0, _G, step, 0)


def _sc_gather(xn, idx_flat):
    mesh = plsc.VectorSubcoreMesh(core_axis_name="c", subcore_axis_name="s")
    f = pl.kernel(
        _sc_gather_body,
        out_type=jax.ShapeDtypeStruct((_R, _D), jnp.float32),
        mesh=mesh,
        scratch_types=[
            pltpu.VMEM((1, _CH), jnp.int32),
            pltpu.VMEM((1, _CH, _D), jnp.float32),
            pltpu.SemaphoreType.DMA,
        ],
    )
    return f(xn, idx_flat)


def _route_body(z_ref, xn_ref, out_ref):
    z = z_ref[...]
    xn = xn_ref[...]
    u = jnp.mean(z, axis=1) + xn
    for _ in range(2):
        nrm2 = jnp.sum(u * u, axis=1, keepdims=True)
        squash = nrm2 / (nrm2 + 1.0)
        v = squash * u / jnp.maximum(jnp.sqrt(nrm2), _EPS)
        p = jnp.sum(z * v[:, None, :], axis=2)
        p = jax.nn.softmax(p, axis=1)
        u = jnp.sum(z * p[:, :, None], axis=1) + xn
    out_ref[...] = u


def _route(z3, xn):
    grid = (_NPAD // _NBLK,)
    return pl.pallas_call(
        _route_body,
        grid=grid,
        in_specs=[
            pl.BlockSpec((_NBLK, _M, _D), lambda i: (i, 0, 0)),
            pl.BlockSpec((_NBLK, _D), lambda i: (i, 0)),
        ],
        out_specs=pl.BlockSpec((_NBLK, _D), lambda i: (i, 0)),
        out_shape=jax.ShapeDtypeStruct((_NPAD, _D), jnp.float32),
        compiler_params=pltpu.CompilerParams(
            dimension_semantics=("arbitrary",)),
    )(z3, xn)


def kernel(x, x_nb):
    n, d = x.shape
    x_pad = jnp.pad(x, ((0, _NPAD - n), (0, 0)))
    nb_pad = jnp.pad(x_nb, ((0, _NPAD - n), (0, 0)), constant_values=1)
    xn, idx = _prep(x_pad, nb_pad)
    z = _sc_gather(xn, idx.reshape(-1))
    u = _route(z.reshape(_NPAD, _M, _D), xn)
    return u[:n]


# speedup vs baseline: 1.3714x; 1.1168x over previous
import functools

import jax
import jax.numpy as jnp
from jax import lax
from jax.experimental import pallas as pl
from jax.experimental.pallas import tpu as pltpu
from jax.experimental.pallas import tpu_sc as plsc

_N = 10000
_M = 32
_D = 128
_NC = 2
_NS = 16
_NW = _NC * _NS
_NPAD = 10240
_R = _NPAD * _M
_RW = _R // _NW
_CH = 128
_G = _RW // _CH
_NBLK = 256
_EPS = 1e-12


def _prep_body(x_ref, nb_ref, xn_ref, idx_ref):
    xb = x_ref[...]
    nrm = jnp.sqrt(jnp.sum(xb * xb, axis=1, keepdims=True))
    xn_ref[...] = xb / jnp.maximum(nrm, _EPS)
    idx_ref[...] = nb_ref[...] - 1


def _prep(x_pad, nb_pad):
    return pl.pallas_call(
        _prep_body,
        out_shape=[
            jax.ShapeDtypeStruct((_NPAD, _D), jnp.float32),
            jax.ShapeDtypeStruct((_NPAD, _M), jnp.int32),
        ],
    )(x_pad, nb_pad)


_K = 4


def _sc_gather_body(xn_hbm, idx_hbm, z_hbm, idx_v, zbuf, gsem, wsem):
    wid = lax.axis_index("s") * _NC + lax.axis_index("c")

    pltpu.sync_copy(idx_hbm.at[pl.ds(wid * _G, _G)], idx_v)

    def gather(g, b):
        pltpu.async_copy(xn_hbm.at[idx_v.at[g]], zbuf.at[b], gsem)

    def gather_wait(b):
        pltpu.make_async_copy(xn_hbm.at[idx_v.at[0]], zbuf.at[b], gsem).wait()

    def write(g, b):
        pltpu.async_copy(zbuf.at[b], z_hbm.at[pl.ds(wid * _RW + g * _CH, _CH)],
                         wsem)

    def write_wait(b):
        pltpu.make_async_copy(zbuf.at[b], z_hbm.at[pl.ds(0, _CH)], wsem).wait()

    for b in range(_K):
        gather(b, b)

    def step(g, carry):
        b = lax.rem(g, _K)
        gather_wait(b)
        write(g, b)
        prev = lax.rem(g + _K - 1, _K)

        @pl.when(g > 0)
        def _():
            write_wait(prev)

        @pl.when(jnp.logical_and(g > 0, g + _K - 1 < _G))
        def _():
            gather(g + _K - 1, prev)
        return carry

    lax.fori_loop(0, _G, step, 0)
    write_wait(lax.rem(_G - 1, _K))


def _sc_gather(xn, idx2d):
    mesh = plsc.VectorSubcoreMesh(core_axis_name="c", subcore_axis_name="s")
    f = pl.kernel(
        _sc_gather_body,
        out_type=jax.ShapeDtypeStruct((_R, _D), jnp.float32),
        mesh=mesh,
        scratch_types=[
            pltpu.VMEM((_G, _CH), jnp.int32),
            pltpu.VMEM((_K, _CH, _D), jnp.float32),
            pltpu.SemaphoreType.DMA,
            pltpu.SemaphoreType.DMA,
        ],
    )
    return f(xn, idx2d)


def _route_body(z_ref, xn_ref, out_ref):
    z = z_ref[...]
    xn = xn_ref[...]
    u = jnp.mean(z, axis=1) + xn
    for _ in range(2):
        nrm2 = jnp.sum(u * u, axis=1, keepdims=True)
        squash = nrm2 / (nrm2 + 1.0)
        v = squash * u / jnp.maximum(jnp.sqrt(nrm2), _EPS)
        p = jnp.sum(z * v[:, None, :], axis=2)
        p = jax.nn.softmax(p, axis=1)
        u = jnp.sum(z * p[:, :, None], axis=1) + xn
    out_ref[...] = u


def _route(z3, xn):
    grid = (_NPAD // _NBLK,)
    return pl.pallas_call(
        _route_body,
        grid=grid,
        in_specs=[
            pl.BlockSpec((_NBLK, _M, _D), lambda i: (i, 0, 0)),
            pl.BlockSpec((_NBLK, _D), lambda i: (i, 0)),
        ],
        out_specs=pl.BlockSpec((_NBLK, _D), lambda i: (i, 0)),
        out_shape=jax.ShapeDtypeStruct((_NPAD, _D), jnp.float32),
        compiler_params=pltpu.CompilerParams(
            dimension_semantics=("arbitrary",)),
    )(z3, xn)


def kernel(x, x_nb):
    n, d = x.shape
    x_pad = jnp.pad(x, ((0, _NPAD - n), (0, 0)))
    nb_pad = jnp.pad(x_nb, ((0, _NPAD - n), (0, 0)), constant_values=1)
    xn, idx = _prep(x_pad, nb_pad)
    z = _sc_gather(xn, idx.reshape(_R // _CH, _CH))
    u = _route(z.reshape(_NPAD, _M, _D), xn)
    return u[:n]
